# baseline (speedup 1.0000x reference)
import functools

import jax
import jax.numpy as jnp
from jax import lax
from jax.experimental import pallas as pl
from jax.experimental.pallas import tpu as pltpu
from jax.experimental.pallas import tpu_sc as plsc

NUM_SAMPLES = 32
C_FRAMES = 3
T = 300
H = 224
W = 224
S = 8
ROW = (H * W) // S
B = C_FRAMES * NUM_SAMPLES * S
NC, NS = 2, 16
NW = NC * NS
B_PER_W = B // NW
CHUNK = 8
N_CHUNKS = B_PER_W // CHUNK


def _sc_gather(table, idx):
    mesh = plsc.VectorSubcoreMesh(core_axis_name="c", subcore_axis_name="s")

    @functools.partial(
        pl.kernel,
        mesh=mesh,
        out_type=jax.ShapeDtypeStruct((B, ROW), jnp.float32),
        scratch_types=[
            pltpu.VMEM((B_PER_W,), jnp.int32),
            pltpu.VMEM((CHUNK, ROW), jnp.float32),
            pltpu.SemaphoreType.DMA,
        ],
    )
    def k(table_hbm, idx_hbm, out_hbm, idx_v, rows_v, sem):
        wid = lax.axis_index("s") * NC + lax.axis_index("c")
        base = wid * B_PER_W
        pltpu.sync_copy(idx_hbm.at[pl.ds(base, B_PER_W)], idx_v)

        @pl.loop(0, N_CHUNKS)
        def _(i):
            off = i * CHUNK
            pltpu.async_copy(
                table_hbm.at[idx_v.at[pl.ds(off, CHUNK)]], rows_v, sem
            ).wait()
            pltpu.sync_copy(rows_v, out_hbm.at[pl.ds(base + off, CHUNK)])

    return k(table, idx)


def kernel(x):
    t = x.shape[-3]
    indices = jnp.linspace(0.0, t - 1, NUM_SAMPLES)
    indices = jnp.clip(indices, 0, t - 1).astype(jnp.int32)
    frame_rows = (jnp.arange(C_FRAMES, dtype=jnp.int32) * t)[:, None] + indices[None, :]
    full = (
        frame_rows.reshape(-1)[:, None] * S
        + jnp.arange(S, dtype=jnp.int32)[None, :]
    ).reshape(-1)
    table = x.reshape(C_FRAMES * t * S, ROW)
    out = _sc_gather(table, full)
    return out.reshape(C_FRAMES, NUM_SAMPLES, H, W)
